# Initial kernel scaffold; baseline (speedup 1.0000x reference)
import jax
import jax.numpy as jnp
from jax.experimental import pallas as pl


def kernel(output, mask, index, target):
    raise NotImplementedError("write your pallas kernel here")


# speedup vs baseline: 1.5522x; 1.5522x over previous
import functools

import jax
import jax.numpy as jnp
from jax import lax
from jax.experimental import pallas as pl
from jax.experimental.pallas import tpu as pltpu
from jax.experimental.pallas import tpu_sc as plsc

B, C, H, W, K = 32, 4, 256, 256, 500
HW = H * W
KC = K * C
LANES = 16
NVEC = KC // LANES
EPAD = 2048
NROW = 16
RLEN = EPAD // NROW
NC, NS = 2, 16
LOSS_WEIGHT = 1.0


def _body(table, idxr, maskr, tgtr, outr,
          idx_v, mask_v, gidx_v, rows_v, tgt_v, out_v, sem):
    cid = lax.axis_index("c")
    sid = lax.axis_index("s")
    b = sid * NC + cid

    pltpu.sync_copy(idxr.at[b], idx_v)
    pltpu.sync_copy(maskr, mask_v)
    pltpu.sync_copy(tgtr.at[b], tgt_v)

    iota = lax.iota(jnp.int32, LANES)
    lane_k = lax.shift_right_logical(iota, 2)
    lane_c = jnp.bitwise_and(iota, 3)
    chan_base = (b * C + lane_c) * HW

    def build(t, carry):
        kidx = jnp.minimum(t * 4 + lane_k, K - 1)
        iv = plsc.load_gather(idx_v, [kidx])
        r = lax.shift_right_logical(t, 3)
        col = lax.shift_left(jnp.bitwise_and(t, 7), 4)
        gidx_v[r, pl.ds(col, LANES)] = chan_base + iv
        return carry

    lax.fori_loop(0, EPAD // LANES, build, 0)

    copies = []
    for r in range(NROW):
        copies.append(
            pltpu.async_copy(table.at[gidx_v.at[r]],
                             rows_v.at[pl.ds(r * RLEN, RLEN)], sem))

    def msum(j, acc):
        return acc + mask_v[pl.ds(j * LANES, LANES)]

    acc = lax.fori_loop(0, (B * K) // LANES, msum,
                        jnp.zeros((LANES,), jnp.int32))
    total = jnp.sum(acc.astype(jnp.float32))
    inv_v = LOSS_WEIGHT / (jnp.broadcast_to(total, (LANES,)) * C + 1e-4)

    for cp in copies:
        cp.wait()

    def comp(t, carry):
        g = rows_v[pl.ds(t * LANES, LANES)]
        tg = tgt_v[pl.ds(t * LANES, LANES)]
        me = plsc.load_gather(mask_v, [b * K + t * 4 + lane_k])
        out_v[pl.ds(t * LANES, LANES)] = (
            jnp.abs(g - tg) * me.astype(jnp.float32) * inv_v)
        return carry

    lax.fori_loop(0, NVEC, comp, 0)
    pltpu.sync_copy(out_v, outr.at[b])


@jax.jit
def _gather_l1(table, idx, msk, tgt):
    return pl.kernel(
        _body,
        out_type=jax.ShapeDtypeStruct((B, KC), jnp.float32),
        mesh=plsc.VectorSubcoreMesh(core_axis_name="c", subcore_axis_name="s"),
        compiler_params=pltpu.CompilerParams(needs_layout_passes=False),
        scratch_types=[
            pltpu.VMEM((K,), jnp.int32),
            pltpu.VMEM((B * K,), jnp.int32),
            pltpu.VMEM((NROW, RLEN), jnp.int32),
            pltpu.VMEM((EPAD,), jnp.float32),
            pltpu.VMEM((KC,), jnp.float32),
            pltpu.VMEM((KC,), jnp.float32),
            pltpu.SemaphoreType.DMA,
        ],
    )(table, idx, msk, tgt)


def kernel(output, mask, index, target):
    table = output.reshape(-1)
    idx = index.astype(jnp.int32)
    msk = mask.astype(jnp.int32).reshape(-1)
    tgt = target.reshape(B, KC)
    out = _gather_l1(table, idx, msk, tgt)
    return out.reshape(B, K, C)


# speedup vs baseline: 2.2039x; 1.4199x over previous
import functools

import jax
import jax.numpy as jnp
from jax import lax
from jax.experimental import pallas as pl
from jax.experimental.pallas import tpu as pltpu
from jax.experimental.pallas import tpu_sc as plsc

B, C, H, W, K = 32, 4, 256, 256, 500
HW = H * W
KC = K * C
LANES = 16
KPAD = 512
NKV = KPAD // LANES
NC, NS = 2, 16
LOSS_WEIGHT = 1.0


def _body(table, idxr, maskr, tgtr, outr,
          idx_v, mask_a, mask_b, mask_me, plane_v, tgt_v, out_v,
          tot_sm, sem):
    cid = lax.axis_index("c")
    sid = lax.axis_index("s")
    b = sid * NC + cid

    pltpu.sync_copy(idxr.at[b], idx_v)
    pltpu.sync_copy(tgtr.at[b], tgt_v)
    pltpu.sync_copy(maskr.at[sid * 2], mask_a)
    pltpu.sync_copy(maskr.at[sid * 2 + 1], mask_b)
    pltpu.sync_copy(maskr.at[b], mask_me)

    def msum(j, acc):
        sl = pl.ds(j * LANES, LANES)
        return acc + mask_a[sl] + mask_b[sl]

    acc = lax.fori_loop(0, NKV, msum, jnp.zeros((LANES,), jnp.int32))
    part = jnp.sum(acc)

    @pl.when(sid == 0)
    def _():
        tot_sm[0] = 0

    plsc.subcore_barrier()
    plsc.fetch_and_add(tot_sm.at[0], part, subcore_id=0)
    plsc.subcore_barrier()
    total = plsc.fetch_and_add(tot_sm.at[0], 0, subcore_id=0)

    inv_v = LOSS_WEIGHT / (
        jnp.broadcast_to(total, (LANES,)).astype(jnp.float32) * C + 1e-4)

    iota = lax.iota(jnp.int32, LANES)

    for cc in range(C):
        pltpu.sync_copy(table.at[b, cc], plane_v)

        def comp(v, carry):
            kvec = jnp.minimum(v * LANES + iota, K - 1)
            iv = plsc.load_gather(idx_v, [kvec])
            h = lax.shift_right_logical(iv, 8)
            w = jnp.bitwise_and(iv, 255)
            g = plsc.load_gather(plane_v, [h, w])
            evec = kvec * C + cc
            tg = plsc.load_gather(tgt_v, [evec])
            me = plsc.load_gather(mask_me, [kvec])
            res = jnp.abs(g - tg) * me.astype(jnp.float32) * inv_v
            plsc.store_scatter(out_v, [evec], res)
            return carry

        lax.fori_loop(0, NKV, comp, 0)

    pltpu.sync_copy(out_v, outr.at[b])


@jax.jit
def _gather_l1(table, idx, msk, tgt):
    return pl.kernel(
        _body,
        out_type=jax.ShapeDtypeStruct((B, KC), jnp.float32),
        mesh=plsc.VectorSubcoreMesh(core_axis_name="c", subcore_axis_name="s"),
        compiler_params=pltpu.CompilerParams(needs_layout_passes=False),
        scratch_types=[
            pltpu.VMEM((K,), jnp.int32),
            pltpu.VMEM((KPAD,), jnp.int32),
            pltpu.VMEM((KPAD,), jnp.int32),
            pltpu.VMEM((KPAD,), jnp.int32),
            pltpu.VMEM((H, W), jnp.float32),
            pltpu.VMEM((KC,), jnp.float32),
            pltpu.VMEM((KC,), jnp.float32),
            pltpu.SMEM((1,), jnp.int32),
            pltpu.SemaphoreType.DMA,
        ],
    )(table, idx, msk, tgt)


def kernel(output, mask, index, target):
    idx = index.astype(jnp.int32)
    msk = jnp.pad(mask.astype(jnp.int32), ((0, 0), (0, KPAD - K)))
    tgt = target.reshape(B, KC)
    out = _gather_l1(output, idx, msk, tgt)
    return out.reshape(B, K, C)


# speedup vs baseline: 2.2677x; 1.0289x over previous
import functools

import jax
import jax.numpy as jnp
from jax import lax
from jax.experimental import pallas as pl
from jax.experimental.pallas import tpu as pltpu
from jax.experimental.pallas import tpu_sc as plsc

B, C, H, W, K = 32, 4, 256, 256, 500
HW = H * W
LANES = 16
NKV = (K + LANES - 1) // LANES
NC, NS = 2, 16
LOSS_WEIGHT = 1.0


def _body(table, idxr, maskr, tgtr, outr,
          idx_v, mask_a, mask_b, mask_me, plane_v, tgt_v, out_v,
          tot_sm, sem):
    cid = lax.axis_index("c")
    sid = lax.axis_index("s")
    b = sid * NC + cid

    pltpu.sync_copy(idxr.at[b], idx_v)
    pltpu.sync_copy(tgtr.at[b], tgt_v)
    pltpu.sync_copy(maskr.at[sid * 2], mask_a)
    pltpu.sync_copy(maskr.at[sid * 2 + 1], mask_b)
    pltpu.sync_copy(maskr.at[b], mask_me)

    iota = lax.iota(jnp.int32, LANES)

    def msum(j, acc):
        kv = j * LANES + iota
        kvec = jnp.minimum(kv, K - 1)
        valid = kv < K
        ga = plsc.load_gather(mask_a, [kvec])
        gb = plsc.load_gather(mask_b, [kvec])
        zero = jnp.zeros((LANES,), jnp.int32)
        return acc + jnp.where(valid, ga + gb, zero)

    acc = lax.fori_loop(0, NKV, msum, jnp.zeros((LANES,), jnp.int32))
    part = jnp.sum(acc)

    @pl.when(sid == 0)
    def _():
        tot_sm[0] = 0

    plsc.subcore_barrier()
    plsc.fetch_and_add(tot_sm.at[0], part, subcore_id=0)
    plsc.subcore_barrier()
    total = plsc.fetch_and_add(tot_sm.at[0], 0, subcore_id=0)

    inv_v = LOSS_WEIGHT / (
        jnp.broadcast_to(total, (LANES,)).astype(jnp.float32) * C + 1e-4)

    for cc in range(C):
        pltpu.sync_copy(table.at[b, cc], plane_v)
        cvec = jnp.broadcast_to(jnp.int32(cc), (LANES,))

        def comp(v, carry):
            kvec = jnp.minimum(v * LANES + iota, K - 1)
            iv = plsc.load_gather(idx_v, [kvec])
            h = lax.shift_right_logical(iv, 8)
            w = jnp.bitwise_and(iv, 255)
            g = plsc.load_gather(plane_v, [h, w])
            evec = kvec * C + cvec
            tg = plsc.load_gather(tgt_v, [evec])
            me = plsc.load_gather(mask_me, [kvec])
            res = jnp.abs(g - tg) * me.astype(jnp.float32) * inv_v
            plsc.store_scatter(out_v, [evec], res)
            return carry

        lax.fori_loop(0, NKV, comp, 0)

    pltpu.sync_copy(out_v, outr.at[b])


@jax.jit
def _gather_l1(table, idx, msk, tgt):
    return pl.kernel(
        _body,
        out_type=jax.ShapeDtypeStruct((B, K * C), jnp.float32),
        mesh=plsc.VectorSubcoreMesh(core_axis_name="c", subcore_axis_name="s"),
        compiler_params=pltpu.CompilerParams(needs_layout_passes=False),
        scratch_types=[
            pltpu.VMEM((K,), jnp.int32),
            pltpu.VMEM((K,), jnp.int32),
            pltpu.VMEM((K,), jnp.int32),
            pltpu.VMEM((K,), jnp.int32),
            pltpu.VMEM((H, W), jnp.float32),
            pltpu.VMEM((K * C,), jnp.float32),
            pltpu.VMEM((K * C,), jnp.float32),
            pltpu.SMEM((1,), jnp.int32),
            pltpu.SemaphoreType.DMA,
        ],
    )(table, idx, msk, tgt)


def kernel(output, mask, index, target):
    idx = index.astype(jnp.int32)
    msk = mask.astype(jnp.int32)
    tgt = target.reshape(B, K * C)
    out = _gather_l1(output, idx, msk, tgt)
    return out.reshape(B, K, C)


# speedup vs baseline: 2.2759x; 1.0036x over previous
import functools

import jax
import jax.numpy as jnp
from jax import lax
from jax.experimental import pallas as pl
from jax.experimental.pallas import tpu as pltpu
from jax.experimental.pallas import tpu_sc as plsc

B, C, H, W, K = 32, 4, 256, 256, 500
HW = H * W
LANES = 16
KPAD = 512
NKV = KPAD // LANES
HH = H // 2
NCHUNK = 2 * C
NC, NS = 2, 16
LOSS_WEIGHT = 1.0


def _body(table, idxr, maskr, tgtr, outr,
          idx_v, mask_a, mask_b, mask_me, buf_a, buf_b, tgt_v, out_v,
          h_v, w_v, e0_v, mf_v, tot_sm, sem):
    cid = lax.axis_index("c")
    sid = lax.axis_index("s")
    b = sid * NC + cid

    bufs = (buf_a, buf_b)

    def chunk_src(i):
        return table.at[b, i // 2, pl.ds((i % 2) * HH, HH)]

    cp = pltpu.async_copy(chunk_src(0), bufs[0], sem)
    pltpu.sync_copy(idxr.at[b], idx_v)
    pltpu.sync_copy(tgtr.at[b], tgt_v)
    pltpu.sync_copy(maskr.at[sid * 2], mask_a)
    pltpu.sync_copy(maskr.at[sid * 2 + 1], mask_b)
    pltpu.sync_copy(maskr.at[b], mask_me)

    iota = lax.iota(jnp.int32, LANES)

    def msum(j, acc):
        kv = j * LANES + iota
        kvec = jnp.minimum(kv, K - 1)
        valid = kv < K
        ga = plsc.load_gather(mask_a, [kvec])
        gb = plsc.load_gather(mask_b, [kvec])
        zero = jnp.zeros((LANES,), jnp.int32)
        return acc + jnp.where(valid, ga + gb, zero)

    acc = lax.fori_loop(0, NKV, msum, jnp.zeros((LANES,), jnp.int32))
    part = jnp.sum(acc)

    @pl.when(sid == 0)
    def _():
        tot_sm[0] = 0

    plsc.subcore_barrier()
    plsc.fetch_and_add(tot_sm.at[0], part, subcore_id=0)
    plsc.subcore_barrier()
    total = plsc.fetch_and_add(tot_sm.at[0], 0, subcore_id=0)

    inv_v = LOSS_WEIGHT / (
        jnp.broadcast_to(total, (LANES,)).astype(jnp.float32) * C + 1e-4)

    def prep(v, carry):
        sl = pl.ds(v * LANES, LANES)
        kvec = jnp.minimum(v * LANES + iota, K - 1)
        iv = plsc.load_gather(idx_v, [kvec])
        me = plsc.load_gather(mask_me, [kvec])
        h_v[sl] = lax.shift_right_logical(iv, 8)
        w_v[sl] = jnp.bitwise_and(iv, 255)
        e0_v[sl] = kvec * C
        mf_v[sl] = me.astype(jnp.float32) * inv_v
        return carry

    lax.fori_loop(0, NKV, prep, 0)

    for i in range(NCHUNK):
        cur = bufs[i % 2]
        cp.wait()
        if i + 1 < NCHUNK:
            cp = pltpu.async_copy(chunk_src(i + 1), bufs[(i + 1) % 2], sem)
        cc = i // 2
        lo = (i % 2) * HH

        def comp(v, carry):
            sl = pl.ds(v * LANES, LANES)
            hl = h_v[sl] - lo
            sel = jnp.logical_and(hl >= 0, hl < HH)
            g = plsc.load_gather(cur, [jnp.bitwise_and(hl, HH - 1), w_v[sl]])
            ev = e0_v[sl] + cc
            tg = plsc.load_gather(tgt_v, [ev])
            res = jnp.abs(g - tg) * mf_v[sl]
            plsc.store_scatter(out_v, [ev], res, mask=sel)
            return carry

        lax.fori_loop(0, NKV, comp, 0)

    pltpu.sync_copy(out_v, outr.at[b])


@jax.jit
def _gather_l1(table, idx, msk, tgt):
    return pl.kernel(
        _body,
        out_type=jax.ShapeDtypeStruct((B, K * C), jnp.float32),
        mesh=plsc.VectorSubcoreMesh(core_axis_name="c", subcore_axis_name="s"),
        compiler_params=pltpu.CompilerParams(needs_layout_passes=False),
        scratch_types=[
            pltpu.VMEM((K,), jnp.int32),
            pltpu.VMEM((K,), jnp.int32),
            pltpu.VMEM((K,), jnp.int32),
            pltpu.VMEM((K,), jnp.int32),
            pltpu.VMEM((HH, W), jnp.float32),
            pltpu.VMEM((HH, W), jnp.float32),
            pltpu.VMEM((K * C,), jnp.float32),
            pltpu.VMEM((K * C,), jnp.float32),
            pltpu.VMEM((KPAD,), jnp.int32),
            pltpu.VMEM((KPAD,), jnp.int32),
            pltpu.VMEM((KPAD,), jnp.int32),
            pltpu.VMEM((KPAD,), jnp.float32),
            pltpu.SMEM((1,), jnp.int32),
            pltpu.SemaphoreType.DMA,
        ],
    )(table, idx, msk, tgt)


def kernel(output, mask, index, target):
    idx = index.astype(jnp.int32)
    msk = mask.astype(jnp.int32)
    tgt = target.reshape(B, K * C)
    out = _gather_l1(output, idx, msk, tgt)
    return out.reshape(B, K, C)


# speedup vs baseline: 2.5122x; 1.1038x over previous
import functools

import jax
import jax.numpy as jnp
from jax import lax
from jax.experimental import pallas as pl
from jax.experimental.pallas import tpu as pltpu
from jax.experimental.pallas import tpu_sc as plsc

B, C, H, W, K = 32, 4, 256, 256, 500
HW = H * W
LANES = 16
KPAD = 512
NKV = KPAD // LANES
NQ = 4
HH = H // NQ
NCHUNK = NQ * C
NBUF = 4
NC, NS = 2, 16
LOSS_WEIGHT = 1.0


def _body(table, idxr, maskr, tgtr, outr,
          idx_v, mask_a, mask_b, mask_me, buf_a, buf_b, buf_c, buf_d,
          tgt_v, out_v, h_v, w_v, e0_v, mf_v, tot_sm, sem):
    cid = lax.axis_index("c")
    sid = lax.axis_index("s")
    b = sid * NC + cid

    bufs = (buf_a, buf_b, buf_c, buf_d)

    def chunk_src(i):
        return table.at[b, i // NQ, pl.ds((i % NQ) * HH, HH)]

    cps = [pltpu.async_copy(chunk_src(i), bufs[i], sem)
           for i in range(NBUF - 1)]
    pltpu.sync_copy(idxr.at[b], idx_v)
    pltpu.sync_copy(tgtr.at[b], tgt_v)
    pltpu.sync_copy(maskr.at[sid * 2], mask_a)
    pltpu.sync_copy(maskr.at[sid * 2 + 1], mask_b)
    pltpu.sync_copy(maskr.at[b], mask_me)

    iota = lax.iota(jnp.int32, LANES)

    def msum(j, acc):
        kv = j * LANES + iota
        kvec = jnp.minimum(kv, K - 1)
        valid = kv < K
        ga = plsc.load_gather(mask_a, [kvec])
        gb = plsc.load_gather(mask_b, [kvec])
        zero = jnp.zeros((LANES,), jnp.int32)
        return acc + jnp.where(valid, ga + gb, zero)

    acc = lax.fori_loop(0, NKV, msum, jnp.zeros((LANES,), jnp.int32))
    part = jnp.sum(acc)

    @pl.when(sid == 0)
    def _():
        tot_sm[0] = 0

    plsc.subcore_barrier()
    plsc.fetch_and_add(tot_sm.at[0], part, subcore_id=0)
    plsc.subcore_barrier()
    total = plsc.fetch_and_add(tot_sm.at[0], 0, subcore_id=0)

    inv_v = LOSS_WEIGHT / (
        jnp.broadcast_to(total, (LANES,)).astype(jnp.float32) * C + 1e-4)

    def prep(v, carry):
        sl = pl.ds(v * LANES, LANES)
        kvec = jnp.minimum(v * LANES + iota, K - 1)
        iv = plsc.load_gather(idx_v, [kvec])
        me = plsc.load_gather(mask_me, [kvec])
        h_v[sl] = lax.shift_right_logical(iv, 8)
        w_v[sl] = jnp.bitwise_and(iv, 255)
        e0_v[sl] = kvec * C
        mf_v[sl] = me.astype(jnp.float32) * inv_v
        return carry

    lax.fori_loop(0, NKV, prep, 0)

    for i in range(NCHUNK):
        cur = bufs[i % NBUF]
        cps[i].wait()
        if i + NBUF - 1 < NCHUNK:
            cps.append(pltpu.async_copy(
                chunk_src(i + NBUF - 1), bufs[(i + NBUF - 1) % NBUF], sem))
        cc = i // NQ
        lo = (i % NQ) * HH

        def comp(v, carry):
            sl = pl.ds(v * LANES, LANES)
            hl = h_v[sl] - lo
            sel = jnp.logical_and(hl >= 0, hl < HH)
            g = plsc.load_gather(cur, [jnp.bitwise_and(hl, HH - 1), w_v[sl]])
            ev = e0_v[sl] + cc
            tg = plsc.load_gather(tgt_v, [ev])
            res = jnp.abs(g - tg) * mf_v[sl]
            plsc.store_scatter(out_v, [ev], res, mask=sel)
            return carry

        lax.fori_loop(0, NKV, comp, 0)

    pltpu.sync_copy(out_v, outr.at[b])


@jax.jit
def _gather_l1(table, idx, msk, tgt):
    return pl.kernel(
        _body,
        out_type=jax.ShapeDtypeStruct((B, K * C), jnp.float32),
        mesh=plsc.VectorSubcoreMesh(core_axis_name="c", subcore_axis_name="s"),
        compiler_params=pltpu.CompilerParams(needs_layout_passes=False),
        scratch_types=[
            pltpu.VMEM((K,), jnp.int32),
            pltpu.VMEM((K,), jnp.int32),
            pltpu.VMEM((K,), jnp.int32),
            pltpu.VMEM((K,), jnp.int32),
            pltpu.VMEM((HH, W), jnp.float32),
            pltpu.VMEM((HH, W), jnp.float32),
            pltpu.VMEM((HH, W), jnp.float32),
            pltpu.VMEM((HH, W), jnp.float32),
            pltpu.VMEM((K * C,), jnp.float32),
            pltpu.VMEM((K * C,), jnp.float32),
            pltpu.VMEM((KPAD,), jnp.int32),
            pltpu.VMEM((KPAD,), jnp.int32),
            pltpu.VMEM((KPAD,), jnp.int32),
            pltpu.VMEM((KPAD,), jnp.float32),
            pltpu.SMEM((1,), jnp.int32),
            pltpu.SemaphoreType.DMA,
        ],
    )(table, idx, msk, tgt)


def kernel(output, mask, index, target):
    idx = index.astype(jnp.int32)
    msk = mask.astype(jnp.int32)
    tgt = target.reshape(B, K * C)
    out = _gather_l1(output, idx, msk, tgt)
    return out.reshape(B, K, C)


# speedup vs baseline: 2.5245x; 1.0049x over previous
import functools

import jax
import jax.numpy as jnp
from jax import lax
from jax.experimental import pallas as pl
from jax.experimental.pallas import tpu as pltpu
from jax.experimental.pallas import tpu_sc as plsc

B, C, H, W, K = 32, 4, 256, 256, 500
HW = H * W
LANES = 16
KPAD = 512
NKV = KPAD // LANES
NQ = 4
HH = H // NQ
NCHUNK = NQ * C
NBUF = 6
NC, NS = 2, 16
LOSS_WEIGHT = 1.0


def _body(table, idxr, maskr, tgtr, outr,
          idx_v, mask_a, mask_b, mask_me, buf_a, buf_b, buf_c, buf_d,
          buf_e, buf_f, tgt_v, out_v, h_v, w_v, e0_v, mf_v, tot_sm, sem):
    cid = lax.axis_index("c")
    sid = lax.axis_index("s")
    b = sid * NC + cid

    bufs = (buf_a, buf_b, buf_c, buf_d, buf_e, buf_f)

    def chunk_src(i):
        return table.at[b, i // NQ, pl.ds((i % NQ) * HH, HH)]

    cps = [pltpu.async_copy(chunk_src(i), bufs[i], sem)
           for i in range(NBUF - 1)]
    pltpu.sync_copy(idxr.at[b], idx_v)
    pltpu.sync_copy(tgtr.at[b], tgt_v)
    pltpu.sync_copy(maskr.at[sid * 2], mask_a)
    pltpu.sync_copy(maskr.at[sid * 2 + 1], mask_b)
    pltpu.sync_copy(maskr.at[b], mask_me)

    iota = lax.iota(jnp.int32, LANES)

    def msum(j, acc):
        kv = j * LANES + iota
        kvec = jnp.minimum(kv, K - 1)
        valid = kv < K
        ga = plsc.load_gather(mask_a, [kvec])
        gb = plsc.load_gather(mask_b, [kvec])
        zero = jnp.zeros((LANES,), jnp.int32)
        return acc + jnp.where(valid, ga + gb, zero)

    acc = lax.fori_loop(0, NKV, msum, jnp.zeros((LANES,), jnp.int32))
    part = jnp.sum(acc)

    @pl.when(sid == 0)
    def _():
        tot_sm[0] = 0

    plsc.subcore_barrier()
    plsc.fetch_and_add(tot_sm.at[0], part, subcore_id=0)
    plsc.subcore_barrier()
    total = plsc.fetch_and_add(tot_sm.at[0], 0, subcore_id=0)

    inv_v = LOSS_WEIGHT / (
        jnp.broadcast_to(total, (LANES,)).astype(jnp.float32) * C + 1e-4)

    def prep(v, carry):
        sl = pl.ds(v * LANES, LANES)
        kvec = jnp.minimum(v * LANES + iota, K - 1)
        iv = plsc.load_gather(idx_v, [kvec])
        me = plsc.load_gather(mask_me, [kvec])
        h_v[sl] = lax.shift_right_logical(iv, 8)
        w_v[sl] = jnp.bitwise_and(iv, 255)
        e0_v[sl] = kvec * C
        mf_v[sl] = me.astype(jnp.float32) * inv_v
        return carry

    lax.fori_loop(0, NKV, prep, 0)

    for i in range(NCHUNK):
        cur = bufs[i % NBUF]
        cps[i].wait()
        if i + NBUF - 1 < NCHUNK:
            cps.append(pltpu.async_copy(
                chunk_src(i + NBUF - 1), bufs[(i + NBUF - 1) % NBUF], sem))
        cc = i // NQ
        lo = (i % NQ) * HH

        def comp(v, carry):
            sl = pl.ds(v * LANES, LANES)
            hl = h_v[sl] - lo
            sel = jnp.logical_and(hl >= 0, hl < HH)
            g = plsc.load_gather(cur, [jnp.bitwise_and(hl, HH - 1), w_v[sl]])
            ev = e0_v[sl] + cc
            tg = plsc.load_gather(tgt_v, [ev])
            res = jnp.abs(g - tg) * mf_v[sl]
            plsc.store_scatter(out_v, [ev], res, mask=sel)
            return carry

        lax.fori_loop(0, NKV, comp, 0)

    pltpu.sync_copy(out_v, outr.at[b])


@jax.jit
def _gather_l1(table, idx, msk, tgt):
    return pl.kernel(
        _body,
        out_type=jax.ShapeDtypeStruct((B, K * C), jnp.float32),
        mesh=plsc.VectorSubcoreMesh(core_axis_name="c", subcore_axis_name="s"),
        compiler_params=pltpu.CompilerParams(needs_layout_passes=False),
        scratch_types=[
            pltpu.VMEM((K,), jnp.int32),
            pltpu.VMEM((K,), jnp.int32),
            pltpu.VMEM((K,), jnp.int32),
            pltpu.VMEM((K,), jnp.int32),
            pltpu.VMEM((HH, W), jnp.float32),
            pltpu.VMEM((HH, W), jnp.float32),
            pltpu.VMEM((HH, W), jnp.float32),
            pltpu.VMEM((HH, W), jnp.float32),
            pltpu.VMEM((HH, W), jnp.float32),
            pltpu.VMEM((HH, W), jnp.float32),
            pltpu.VMEM((K * C,), jnp.float32),
            pltpu.VMEM((K * C,), jnp.float32),
            pltpu.VMEM((KPAD,), jnp.int32),
            pltpu.VMEM((KPAD,), jnp.int32),
            pltpu.VMEM((KPAD,), jnp.int32),
            pltpu.VMEM((KPAD,), jnp.float32),
            pltpu.SMEM((1,), jnp.int32),
            pltpu.SemaphoreType.DMA,
        ],
    )(table, idx, msk, tgt)


def kernel(output, mask, index, target):
    idx = index.astype(jnp.int32)
    msk = mask.astype(jnp.int32)
    tgt = target.reshape(B, K * C)
    out = _gather_l1(output, idx, msk, tgt)
    return out.reshape(B, K, C)
